# Initial kernel scaffold; baseline (speedup 1.0000x reference)
import jax
import jax.numpy as jnp
from jax.experimental import pallas as pl


def kernel(x, table):
    raise NotImplementedError("write your pallas kernel here")


# speedup vs baseline: 1.7982x; 1.7982x over previous
import functools

import jax
import jax.numpy as jnp
from jax import lax
from jax.experimental import pallas as pl
from jax.experimental.pallas import tpu as pltpu
from jax.experimental.pallas import tpu_sc as plsc

NC = 2
NS = 16
NW = NC * NS

SUB = 128
K = 4


def _build_gather(n_groups: int, d: int):
  groups_per_w = n_groups // NW
  steps = groups_per_w // K

  mesh = plsc.VectorSubcoreMesh(core_axis_name="c", subcore_axis_name="s")

  @functools.partial(
      pl.kernel,
      out_type=jax.ShapeDtypeStruct((n_groups, SUB, d), jnp.float32),
      mesh=mesh,
      compiler_params=pltpu.CompilerParams(use_tc_tiling_on_sc=False),
      scratch_types=[
          pltpu.VMEM((K, SUB), jnp.int32),
          pltpu.VMEM((K, SUB, d), jnp.float32),
          pltpu.SemaphoreType.DMA,
      ],
  )
  def gather_kernel(table_hbm, idx_hbm, out_hbm, idx_v, rows_v, sem):
    wid = lax.axis_index("s") * NC + lax.axis_index("c")
    base = wid * groups_per_w

    def step(i, carry):
      rb = base + i * K
      pltpu.sync_copy(idx_hbm.at[pl.ds(rb, K)], idx_v)
      copies = [
          pltpu.async_copy(table_hbm.at[idx_v.at[j]], rows_v.at[j], sem)
          for j in range(K)
      ]
      for cp in copies:
        cp.wait()
      pltpu.sync_copy(rows_v, out_hbm.at[pl.ds(rb, K)])
      return carry

    lax.fori_loop(0, steps, step, 0)

  return gather_kernel


def kernel(x, table):
  b, h = x.shape
  v, d = table.shape
  n = b * h
  assert n % (NW * K * SUB) == 0
  idx = x.reshape(n // SUB, SUB).astype(jnp.int32)
  out = _build_gather(n // SUB, d)(table, idx)
  return out.reshape(b, h, d)


# speedup vs baseline: 1.8737x; 1.0420x over previous
import functools

import jax
import jax.numpy as jnp
from jax import lax
from jax.experimental import pallas as pl
from jax.experimental.pallas import tpu as pltpu
from jax.experimental.pallas import tpu_sc as plsc

NC = 2
NS = 16
NW = NC * NS

SUB = 128
K = 5
NBUF = 2


def _build_gather(n_groups: int, d: int):
  groups_per_w = n_groups // NW
  steps = groups_per_w // K
  outer = steps // NBUF

  mesh = plsc.VectorSubcoreMesh(core_axis_name="c", subcore_axis_name="s")

  @functools.partial(
      pl.kernel,
      out_type=jax.ShapeDtypeStruct((n_groups, SUB, d), jnp.float32),
      mesh=mesh,
      compiler_params=pltpu.CompilerParams(use_tc_tiling_on_sc=False),
      scratch_types=[
          pltpu.VMEM((NBUF, K, SUB), jnp.int32),
          pltpu.VMEM((NBUF, K, SUB, d), jnp.float32),
          pltpu.SemaphoreType.DMA((NBUF,)),
          pltpu.SemaphoreType.DMA((NBUF,)),
          pltpu.SemaphoreType.DMA((NBUF,)),
      ],
  )
  def gather_kernel(table_hbm, idx_hbm, out_hbm, idx_v, rows_v,
                    sem_i, sem_g, sem_w):
    wid = lax.axis_index("s") * NC + lax.axis_index("c")
    base = wid * groups_per_w

    for b in range(NBUF):
      pltpu.async_copy(idx_hbm.at[pl.ds(base + b * K, K)], idx_v.at[b],
                       sem_i.at[b])

    @pl.loop(0, outer)
    def _(t):
      for b in range(NBUF):
        g = t * NBUF + b
        rb = base + g * K
        pltpu.make_async_copy(idx_hbm.at[pl.ds(rb, K)], idx_v.at[b],
                              sem_i.at[b]).wait()
        @pl.when(t >= 1)
        def _():
          pltpu.make_async_copy(rows_v.at[b],
                                out_hbm.at[pl.ds(rb - NBUF * K, K)],
                                sem_w.at[b]).wait()
        copies = [
            pltpu.async_copy(table_hbm.at[idx_v.at[b, j]], rows_v.at[b, j],
                             sem_g.at[b])
            for j in range(K)
        ]
        for cp in copies:
          cp.wait()
        pltpu.async_copy(rows_v.at[b], out_hbm.at[pl.ds(rb, K)], sem_w.at[b])
        @pl.when(t < outer - 1)
        def _():
          pltpu.async_copy(idx_hbm.at[pl.ds(rb + NBUF * K, K)], idx_v.at[b],
                           sem_i.at[b])

    for b in range(NBUF):
      g = (outer - 1) * NBUF + b
      pltpu.make_async_copy(rows_v.at[b],
                            out_hbm.at[pl.ds(base + g * K, K)],
                            sem_w.at[b]).wait()

  return gather_kernel


def kernel(x, table):
  b, h = x.shape
  v, d = table.shape
  n = b * h
  assert n % (NW * K * SUB * NBUF) == 0
  idx = x.reshape(n // SUB, SUB).astype(jnp.int32)
  out = _build_gather(n // SUB, d)(table, idx)
  return out.reshape(b, h, d)
